# baseline (speedup 1.0000x reference)
import numpy as np

import jax
import jax.numpy as jnp
from jax.experimental import pallas as pl
from jax.experimental.pallas import tpu as pltpu

L = 4
S = 4
N_MAX = 10
HID = 32
R_CUT = 5.0
FEAT = L * N_MAX
SH = S * HID
CAND = L * SH

BLOCK = 6400
CHUNKS = BLOCK // 128


def _fast_sin(x):
    n = jnp.floor(x * (2.0 / jnp.pi) + 0.5)
    y = x - n * (jnp.pi / 2.0)
    q = n - 4.0 * jnp.floor(n * 0.25)
    y2 = y * y
    sin_p = y * (1.0 + y2 * (-1.6666667e-1 + y2 * (8.3333310e-3 + y2 * -1.98409e-4)))
    cos_p = 1.0 + y2 * (-0.5 + y2 * (4.16666418e-2 + y2 * -1.388731625e-3))
    use_cos = jnp.logical_or(q == 1.0, q == 3.0)
    val = jnp.where(use_cos, cos_p, sin_p)
    return jnp.where(q >= 2.0, -val, val)


def _unpack_columns(mat):
    cols = [jax.lax.slice(mat, (0, c), (128, c + 1)) for c in range(CHUNKS)]
    return jnp.concatenate(cols, axis=0)


def _rb_mlp_kernel(v_ref, c_ref, w1_ref, w2_ref, w3_ref, w4_ref, out_ref):
    xv = v_ref[0]
    bounds = [0, 6, 12, 18, 25, 31, 37, 43, CHUNKS]
    for hi in range(8):
        c0, c1 = bounds[hi], bounds[hi + 1]
        _part_body(xv[c0:c1], c_ref, w1_ref, w2_ref, w3_ref, w4_ref, out_ref,
                   c0 * 128)


def _part_body(xh, c_ref, w1_ref, w2_ref, w3_ref, w4_ref, out_ref, row0):
    nc = xh.shape[0]
    vt = jnp.transpose(xh)
    st = jnp.floor(vt * (1.0 / 16.0))
    rt = vt - 16.0 * st
    cols_r = [jax.lax.slice(rt, (0, c), (128, c + 1)) for c in range(nc)]
    r = jnp.concatenate(cols_r, axis=0)
    cols_s = [jax.lax.slice(st, (0, c), (128, c + 1)) for c in range(nc)]
    spf = jnp.concatenate(cols_s, axis=0)

    b = r.shape[0]
    r_ = r * (1.0 / R_CUT)

    cz = c_ref[0:1, 0:FEAT]
    clid = c_ref[1:2, 0:FEAT]
    chid = c_ref[2:3, :]

    x = cz * r_
    sinc = _fast_sin(x) / x
    r2 = r_ * r_
    env = jnp.where(clid == 0.0, 1.0,
          jnp.where(clid == 1.0, r_,
          jnp.where(clid == 2.0, r2, r2 * r_)))
    rf = sinc * env

    def silu_h(u):
        return u + u * jnp.tanh(u)

    def dot(a, w):
        return jnp.dot(a, w, preferred_element_type=jnp.float32)

    m128 = chid == spf

    cand = dot(rf, w1_ref[...])

    for l in range(L):
        u1 = jnp.where(m128, jax.lax.slice_in_dim(cand, l * SH, (l + 1) * SH, axis=1), 0.0)
        h = silu_h(u1)
        h = silu_h(dot(h, w2_ref[l]))
        h = silu_h(dot(h, w3_ref[l]))
        out_ref[l, pl.ds(row0, b), :] = dot(h, w4_ref[l])


@jax.jit
def kernel(r, species_neighbor, W1, W2, W3, W4):
    n = r.shape[0]
    block = BLOCK
    grid = n // block

    w1cat = jnp.zeros((FEAT, CAND), jnp.float32)
    w2b = jnp.zeros((L, SH, SH), jnp.float32)
    w3b = jnp.zeros((L, SH, SH), jnp.float32)
    for l in range(L):
        for s in range(S):
            w1cat = w1cat.at[l * N_MAX:(l + 1) * N_MAX,
                             l * SH + s * HID:l * SH + (s + 1) * HID].set(0.5 * W1[l, s])
    for s in range(S):
        w2b = w2b.at[:, s * HID:(s + 1) * HID, s * HID:(s + 1) * HID].set(0.5 * W2[:, s])
        w3b = w3b.at[:, s * HID:(s + 1) * HID, s * HID:(s + 1) * HID].set(0.5 * W3[:, s])
    w4r = W4.reshape(L, SH, N_MAX)

    v = (r + 16.0 * species_neighbor.astype(jnp.float32)).reshape(grid, CHUNKS, 128)

    crows = np.zeros((8, 128), np.float32)
    for l in range(L):
        for nn in range(N_MAX):
            crows[0, l * N_MAX + nn] = np.pi * (nn + 1.0 + 0.5 * l)
            crows[1, l * N_MAX + nn] = l
    crows[2, :] = np.arange(SH) // HID
    crows = jnp.asarray(crows)

    return pl.pallas_call(
        _rb_mlp_kernel,
        grid=(grid,),
        in_specs=[
            pl.BlockSpec((1, CHUNKS, 128), lambda i: (i, 0, 0)),
            pl.BlockSpec((8, 128), lambda i: (0, 0)),
            pl.BlockSpec((FEAT, CAND), lambda i: (0, 0)),
            pl.BlockSpec((L, SH, SH), lambda i: (0, 0, 0)),
            pl.BlockSpec((L, SH, SH), lambda i: (0, 0, 0)),
            pl.BlockSpec((L, SH, N_MAX), lambda i: (0, 0, 0)),
        ],
        out_specs=pl.BlockSpec((L, block, N_MAX), lambda i: (0, i, 0)),
        out_shape=jax.ShapeDtypeStruct((L, n, N_MAX), jnp.float32),
        compiler_params=pltpu.CompilerParams(
            dimension_semantics=("parallel",),
        ),
    )(v, crows, w1cat, w2b, w3b, w4r)


# speedup vs baseline: 1.0291x; 1.0291x over previous
import numpy as np

import jax
import jax.numpy as jnp
from jax.experimental import pallas as pl
from jax.experimental.pallas import tpu as pltpu

L = 4
S = 4
N_MAX = 10
HID = 32
R_CUT = 5.0
FEAT = L * N_MAX
SH = S * HID
CAND = L * SH

BLOCK = 6400
CHUNKS = BLOCK // 128


def _fast_sin(x):
    n = jnp.floor(x * (2.0 / jnp.pi) + 0.5)
    y = x - n * (jnp.pi / 2.0)
    q = n - 4.0 * jnp.floor(n * 0.25)
    y2 = y * y
    sin_p = y * (1.0 + y2 * (-1.6666667e-1 + y2 * (8.3333310e-3 + y2 * -1.98409e-4)))
    cos_p = 1.0 + y2 * (-0.5 + y2 * (4.16666418e-2 + y2 * -1.388731625e-3))
    use_cos = jnp.logical_or(q == 1.0, q == 3.0)
    val = jnp.where(use_cos, cos_p, sin_p)
    return jnp.where(q >= 2.0, -val, val)


def _unpack_columns(mat):
    cols = [jax.lax.slice(mat, (0, c), (128, c + 1)) for c in range(CHUNKS)]
    return jnp.concatenate(cols, axis=0)


def _rb_mlp_kernel(v_ref, c_ref, w1_ref, w2_ref, w3_ref, w4_ref, out_ref):
    xv = v_ref[0]
    bounds = [0, 10, 20, 30, 40, CHUNKS]
    for hi in range(5):
        c0, c1 = bounds[hi], bounds[hi + 1]
        _part_body(xv[c0:c1], c_ref, w1_ref, w2_ref, w3_ref, w4_ref, out_ref,
                   c0 * 128)


def _part_body(xh, c_ref, w1_ref, w2_ref, w3_ref, w4_ref, out_ref, row0):
    nc = xh.shape[0]
    vt = jnp.transpose(xh)
    st = jnp.floor(vt * (1.0 / 16.0))
    rt = vt - 16.0 * st
    cols_r = [jax.lax.slice(rt, (0, c), (128, c + 1)) for c in range(nc)]
    r = jnp.concatenate(cols_r, axis=0)
    cols_s = [jax.lax.slice(st, (0, c), (128, c + 1)) for c in range(nc)]
    spf = jnp.concatenate(cols_s, axis=0)

    b = r.shape[0]
    r_ = r * (1.0 / R_CUT)

    cz = c_ref[0:1, 0:FEAT]
    clid = c_ref[1:2, 0:FEAT]
    chid = c_ref[2:3, :]

    x = cz * r_
    sinc = _fast_sin(x) / x
    r2 = r_ * r_
    env = jnp.where(clid == 0.0, 1.0,
          jnp.where(clid == 1.0, r_,
          jnp.where(clid == 2.0, r2, r2 * r_)))
    rf = sinc * env

    def silu_h(u):
        return u + u * jnp.tanh(u)

    def dot(a, w):
        return jnp.dot(a, w, preferred_element_type=jnp.float32)

    m128 = chid == spf

    cand = dot(rf, w1_ref[...])

    for l in range(L):
        u1 = jnp.where(m128, jax.lax.slice_in_dim(cand, l * SH, (l + 1) * SH, axis=1), 0.0)
        h = silu_h(u1)
        h = silu_h(dot(h, w2_ref[l]))
        h = silu_h(dot(h, w3_ref[l]))
        out_ref[l, pl.ds(row0, b), :] = dot(h, w4_ref[l])


@jax.jit
def kernel(r, species_neighbor, W1, W2, W3, W4):
    n = r.shape[0]
    block = BLOCK
    grid = n // block

    w1cat = jnp.zeros((FEAT, CAND), jnp.float32)
    w2b = jnp.zeros((L, SH, SH), jnp.float32)
    w3b = jnp.zeros((L, SH, SH), jnp.float32)
    for l in range(L):
        for s in range(S):
            w1cat = w1cat.at[l * N_MAX:(l + 1) * N_MAX,
                             l * SH + s * HID:l * SH + (s + 1) * HID].set(0.5 * W1[l, s])
    for s in range(S):
        w2b = w2b.at[:, s * HID:(s + 1) * HID, s * HID:(s + 1) * HID].set(0.5 * W2[:, s])
        w3b = w3b.at[:, s * HID:(s + 1) * HID, s * HID:(s + 1) * HID].set(0.5 * W3[:, s])
    w4r = W4.reshape(L, SH, N_MAX)

    v = (r + 16.0 * species_neighbor.astype(jnp.float32)).reshape(grid, CHUNKS, 128)

    crows = np.zeros((8, 128), np.float32)
    for l in range(L):
        for nn in range(N_MAX):
            crows[0, l * N_MAX + nn] = np.pi * (nn + 1.0 + 0.5 * l)
            crows[1, l * N_MAX + nn] = l
    crows[2, :] = np.arange(SH) // HID
    crows = jnp.asarray(crows)

    return pl.pallas_call(
        _rb_mlp_kernel,
        grid=(grid,),
        in_specs=[
            pl.BlockSpec((1, CHUNKS, 128), lambda i: (i, 0, 0)),
            pl.BlockSpec((8, 128), lambda i: (0, 0)),
            pl.BlockSpec((FEAT, CAND), lambda i: (0, 0)),
            pl.BlockSpec((L, SH, SH), lambda i: (0, 0, 0)),
            pl.BlockSpec((L, SH, SH), lambda i: (0, 0, 0)),
            pl.BlockSpec((L, SH, N_MAX), lambda i: (0, 0, 0)),
        ],
        out_specs=pl.BlockSpec((L, block, N_MAX), lambda i: (0, i, 0)),
        out_shape=jax.ShapeDtypeStruct((L, n, N_MAX), jnp.float32),
        compiler_params=pltpu.CompilerParams(
            dimension_semantics=("parallel",),
        ),
    )(v, crows, w1cat, w2b, w3b, w4r)


# speedup vs baseline: 1.0393x; 1.0100x over previous
import numpy as np

import jax
import jax.numpy as jnp
from jax.experimental import pallas as pl
from jax.experimental.pallas import tpu as pltpu

L = 4
S = 4
N_MAX = 10
HID = 32
R_CUT = 5.0
FEAT = L * N_MAX
SH = S * HID
CAND = L * SH

BLOCK = 6400
CHUNKS = BLOCK // 128


def _fast_sin(x):
    n = jnp.floor(x * (2.0 / jnp.pi) + 0.5)
    y = x - n * (jnp.pi / 2.0)
    q = n - 4.0 * jnp.floor(n * 0.25)
    y2 = y * y
    sin_p = y * (1.0 + y2 * (-1.6666667e-1 + y2 * (8.3333310e-3 + y2 * -1.98409e-4)))
    cos_p = 1.0 + y2 * (-0.5 + y2 * (4.16666418e-2 + y2 * -1.388731625e-3))
    use_cos = jnp.logical_or(q == 1.0, q == 3.0)
    val = jnp.where(use_cos, cos_p, sin_p)
    return jnp.where(q >= 2.0, -val, val)


def _unpack_columns(mat):
    cols = [jax.lax.slice(mat, (0, c), (128, c + 1)) for c in range(CHUNKS)]
    return jnp.concatenate(cols, axis=0)


def _rb_mlp_kernel(v_ref, c_ref, w1_ref, w2_ref, w3_ref, w4_ref, out_ref):
    xv = v_ref[0]
    bounds = [0, 12, 24, 37, CHUNKS]
    for hi in range(4):
        c0, c1 = bounds[hi], bounds[hi + 1]
        _part_body(xv[c0:c1], c_ref, w1_ref, w2_ref, w3_ref, w4_ref, out_ref,
                   c0 * 128)


def _part_body(xh, c_ref, w1_ref, w2_ref, w3_ref, w4_ref, out_ref, row0):
    nc = xh.shape[0]
    vt = jnp.transpose(xh)
    st = jnp.floor(vt * (1.0 / 16.0))
    rt = vt - 16.0 * st
    cols_r = [jax.lax.slice(rt, (0, c), (128, c + 1)) for c in range(nc)]
    r = jnp.concatenate(cols_r, axis=0)
    cols_s = [jax.lax.slice(st, (0, c), (128, c + 1)) for c in range(nc)]
    spf = jnp.concatenate(cols_s, axis=0)

    b = r.shape[0]
    r_ = r * (1.0 / R_CUT)

    cz = c_ref[0:1, 0:FEAT]
    clid = c_ref[1:2, 0:FEAT]
    chid = c_ref[2:3, :]

    x = cz * r_
    sinc = _fast_sin(x) / x
    r2 = r_ * r_
    env = jnp.where(clid == 0.0, 1.0,
          jnp.where(clid == 1.0, r_,
          jnp.where(clid == 2.0, r2, r2 * r_)))
    rf = sinc * env

    def silu_h(u):
        return u + u * jnp.tanh(u)

    def dot(a, w):
        return jnp.dot(a, w, preferred_element_type=jnp.float32)

    m128 = chid == spf

    cand = dot(rf, w1_ref[...])

    for l in range(L):
        u1 = jnp.where(m128, jax.lax.slice_in_dim(cand, l * SH, (l + 1) * SH, axis=1), 0.0)
        h = silu_h(u1)
        h = silu_h(dot(h, w2_ref[l]))
        h = silu_h(dot(h, w3_ref[l]))
        out_ref[l, pl.ds(row0, b), :] = dot(h, w4_ref[l])


@jax.jit
def kernel(r, species_neighbor, W1, W2, W3, W4):
    n = r.shape[0]
    block = BLOCK
    grid = n // block

    w1cat = jnp.zeros((FEAT, CAND), jnp.float32)
    w2b = jnp.zeros((L, SH, SH), jnp.float32)
    w3b = jnp.zeros((L, SH, SH), jnp.float32)
    for l in range(L):
        for s in range(S):
            w1cat = w1cat.at[l * N_MAX:(l + 1) * N_MAX,
                             l * SH + s * HID:l * SH + (s + 1) * HID].set(0.5 * W1[l, s])
    for s in range(S):
        w2b = w2b.at[:, s * HID:(s + 1) * HID, s * HID:(s + 1) * HID].set(0.5 * W2[:, s])
        w3b = w3b.at[:, s * HID:(s + 1) * HID, s * HID:(s + 1) * HID].set(0.5 * W3[:, s])
    w4r = W4.reshape(L, SH, N_MAX)

    v = (r + 16.0 * species_neighbor.astype(jnp.float32)).reshape(grid, CHUNKS, 128)

    crows = np.zeros((8, 128), np.float32)
    for l in range(L):
        for nn in range(N_MAX):
            crows[0, l * N_MAX + nn] = np.pi * (nn + 1.0 + 0.5 * l)
            crows[1, l * N_MAX + nn] = l
    crows[2, :] = np.arange(SH) // HID
    crows = jnp.asarray(crows)

    return pl.pallas_call(
        _rb_mlp_kernel,
        grid=(grid,),
        in_specs=[
            pl.BlockSpec((1, CHUNKS, 128), lambda i: (i, 0, 0)),
            pl.BlockSpec((8, 128), lambda i: (0, 0)),
            pl.BlockSpec((FEAT, CAND), lambda i: (0, 0)),
            pl.BlockSpec((L, SH, SH), lambda i: (0, 0, 0)),
            pl.BlockSpec((L, SH, SH), lambda i: (0, 0, 0)),
            pl.BlockSpec((L, SH, N_MAX), lambda i: (0, 0, 0)),
        ],
        out_specs=pl.BlockSpec((L, block, N_MAX), lambda i: (0, i, 0)),
        out_shape=jax.ShapeDtypeStruct((L, n, N_MAX), jnp.float32),
        compiler_params=pltpu.CompilerParams(
            dimension_semantics=("parallel",),
        ),
    )(v, crows, w1cat, w2b, w3b, w4r)
